# baseline (speedup 1.0000x reference)
import numpy as np
import jax
import jax.numpy as jnp
from jax.experimental import pallas as pl
from jax.experimental.pallas import tpu as pltpu

_C4 = 16
_C = 64
_H = 16
_W = 16
_HW = _H * _W
_MIP = 8
_CR = 4
_BN = 64


def _cad_body(x1_ref, x2_ref, x3_ref, x4_ref, dz_ref, pool_ref, eh_ref,
              ew_ref, w1_ref, w2_ref, c1w_ref, c1b_ref, bns_ref, bnb_ref,
              chw_ref, chb_ref, cww_ref, cwb_ref, o_ref):
    f32 = jnp.float32

    def dot(a, b):
        return jnp.dot(a, b, preferred_element_type=f32)

    tr = lambda a: jnp.swapaxes(a, 0, 1)
    w1t_ref = tr(w1_ref[...])
    w2t_ref = tr(w2_ref[...])
    c1t_ref = tr(c1w_ref[...] * bns_ref[...])
    c1b_ref = tr(bns_ref[...] * c1b_ref[...] + bnb_ref[...])
    cht_ref = tr(chw_ref[...])
    chb_ref = tr(chb_ref[...])
    cwt_ref = tr(cww_ref[...])
    cwb_ref = tr(cwb_ref[...])

    bn = x1_ref.shape[0]
    d = jax.nn.sigmoid(dz_ref[...])
    xs = (x1_ref[...], x2_ref[...], x3_ref[...], x4_ref[...])

    avgs, maxs = [], []
    for x in xs:
        xpd = x + d
        avgs.append(jnp.mean(xpd, axis=2))
        maxs.append(jnp.max(xpd, axis=2))
    ca_avg = jnp.concatenate(avgs, axis=1)
    ca_max = jnp.concatenate(maxs, axis=1)
    w1t = w1t_ref[...]
    hmid = (jnp.maximum(dot(ca_avg, w1t), 0.0)
            + jnp.maximum(dot(ca_max, w1t), 0.0))
    ca = jax.nn.sigmoid(dot(hmid, w2t_ref[...]))

    ip = xs[0] + xs[1] + xs[2] + xs[3] + d
    ypool = dot(ip.reshape(bn * _C4, _HW), pool_ref[...])
    ypt = jnp.swapaxes(ypool.reshape(bn, _C4, _H + _W), 1, 2)
    y = dot(ypt.reshape(bn * (_H + _W), _C4), c1t_ref[...]) + c1b_ref[...]
    y = y * (jnp.clip(y + 3.0, 0.0, 6.0) * (1.0 / 6.0))
    y3 = y.reshape(bn, _H + _W, _MIP)

    pre_h = dot(y3[:, 0:_H, :].reshape(bn * _H, _MIP), cht_ref[...]) \
        + chb_ref[...]
    pre_w = dot(y3[:, _H:, :].reshape(bn * _W, _MIP), cwt_ref[...]) \
        + cwb_ref[...]
    at = jax.nn.sigmoid(jnp.concatenate(
        [pre_h.reshape(bn, _H, _C4), pre_w.reshape(bn, _W, _C4)], axis=1))
    a = jnp.swapaxes(at, 1, 2)

    ah = dot(a[:, :, 0:_H].reshape(bn * _C4, _H), eh_ref[...])
    aw = dot(a[:, :, _H:].reshape(bn * _C4, _W), ew_ref[...])
    hw_a = ip * (ah * aw).reshape(bn, _C4, _HW)

    for i in range(4):
        o_ref[:, i * _C4:(i + 1) * _C4, :] = (
            ca[:, i * _C4:(i + 1) * _C4, None] * (xs[i] + hw_a))


def kernel(x1, x2, x3, x4, distance, ca_w1, ca_w2, c1_w, c1_b, bn_scale,
           bn_shift, ch_w, ch_b, cw_w, cw_b):
    f32 = jnp.float32
    n = x1.shape[0]

    x1f = x1.reshape(n, _C4, _HW)
    x2f = x2.reshape(n, _C4, _HW)
    x3f = x3.reshape(n, _C4, _HW)
    x4f = x4.reshape(n, _C4, _HW)
    dzf = distance.reshape(n, 1, _HW)

    l = np.arange(_HW)
    eh = (l[None, :] // _W == np.arange(_H)[:, None]).astype(np.float32)
    ew = (l[None, :] % _W == np.arange(_W)[:, None]).astype(np.float32)
    pool = np.concatenate([eh.T / _W, ew.T / _H], axis=1)

    bidx = lambda i: (i, 0, 0)
    const = lambda shape: pl.BlockSpec(shape, lambda i: (0,) * len(shape))
    out_flat = pl.pallas_call(
        _cad_body,
        grid=(n // _BN,),
        in_specs=[
            pl.BlockSpec((_BN, _C4, _HW), bidx),
            pl.BlockSpec((_BN, _C4, _HW), bidx),
            pl.BlockSpec((_BN, _C4, _HW), bidx),
            pl.BlockSpec((_BN, _C4, _HW), bidx),
            pl.BlockSpec((_BN, 1, _HW), bidx),
            const((_HW, _H + _W)),
            const((_H, _HW)),
            const((_W, _HW)),
            const((_CR, _C)),
            const((_C, _CR)),
            const((_MIP, _C4)),
            const((_MIP, 1)),
            const((_MIP, 1)),
            const((_MIP, 1)),
            const((_C4, _MIP)),
            const((_C4, 1)),
            const((_C4, _MIP)),
            const((_C4, 1)),
        ],
        out_specs=pl.BlockSpec((_BN, _C, _HW), bidx),
        out_shape=jax.ShapeDtypeStruct((n, _C, _HW), f32),
        compiler_params=pltpu.CompilerParams(
            dimension_semantics=("arbitrary",)),
    )(x1f, x2f, x3f, x4f, dzf, jnp.asarray(pool), jnp.asarray(eh),
      jnp.asarray(ew), ca_w1, ca_w2, c1_w, c1_b, bn_scale, bn_shift,
      ch_w, ch_b, cw_w, cw_b)

    return out_flat.reshape(n, _C, _H, _W)


# speedup vs baseline: 2.3979x; 2.3979x over previous
import jax
import jax.numpy as jnp
from jax.experimental import pallas as pl
from jax.experimental.pallas import tpu as pltpu

_C4 = 16
_C = 64
_H = 16
_W = 16
_HW = _H * _W
_MIP = 8
_CR = 4
_NB = 128


def _cad_body(x1_ref, x2_ref, x3_ref, x4_ref, dz_ref, w1_ref, w2_ref,
              c1w_ref, c1b_ref, bns_ref, bnb_ref, chw_ref, chb_ref,
              cww_ref, cwb_ref, o_ref):
    f32 = jnp.float32

    def dot(a, b):
        return jnp.dot(a, b, preferred_element_type=f32)

    def dotc(w, x):
        return jax.lax.dot_general(w, x, (((1,), (0,)), ((), ())),
                                   preferred_element_type=f32)

    nb = x1_ref.shape[-1]
    d = jax.nn.sigmoid(dz_ref[...])
    xs = (x1_ref[...], x2_ref[...], x3_ref[...], x4_ref[...])

    avg_d = jnp.mean(d, axis=0, keepdims=True)
    avgs, maxs = [], []
    for x in xs:
        x3 = x.reshape(_C4, _HW, nb)
        avgs.append(jnp.mean(x3, axis=1))
        maxs.append(jnp.max(x3 + d[None, :, :], axis=1))
    ca_avg = jnp.concatenate(avgs, axis=0) + avg_d
    ca_max = jnp.concatenate(maxs, axis=0)
    w1 = w1_ref[...]
    hmid = (jnp.maximum(dot(w1, ca_avg), 0.0)
            + jnp.maximum(dot(w1, ca_max), 0.0))
    ca = jax.nn.sigmoid(dot(w2_ref[...], hmid))

    ip = (xs[0] + xs[1] + xs[2] + xs[3]).reshape(_C4, _HW, nb) + d[None, :, :]
    ip4 = ip.reshape(_C4, _H, _W, nb)
    ph = jnp.mean(ip4, axis=2)
    pw = jnp.mean(ip4, axis=1)
    ypool = jnp.concatenate([ph, pw], axis=1)

    c1_wf = c1w_ref[...] * bns_ref[...]
    c1_bf = bns_ref[...] * c1b_ref[...] + bnb_ref[...]
    y = dotc(c1_wf, ypool) + c1_bf[:, :, None]
    y = y * (jnp.clip(y + 3.0, 0.0, 6.0) * (1.0 / 6.0))

    a_h = jax.nn.sigmoid(dotc(chw_ref[...], y[:, 0:_H, :])
                         + chb_ref[...][:, :, None])
    a_w = jax.nn.sigmoid(dotc(cww_ref[...], y[:, _H:, :])
                         + cwb_ref[...][:, :, None])
    g = a_h[:, :, None, :] * a_w[:, None, :, :]
    hw_a = ip4 * g
    hw_a = hw_a.reshape(_C4, _HW, nb)

    for i in range(4):
        blk = (ca[i * _C4:(i + 1) * _C4, :][:, None, :]
               * (xs[i].reshape(_C4, _HW, nb) + hw_a))
        o_ref[i * _C4 * _HW:(i + 1) * _C4 * _HW, :] = blk.reshape(_C4 * _HW, nb)


def kernel(x1, x2, x3, x4, distance, ca_w1, ca_w2, c1_w, c1_b, bn_scale,
           bn_shift, ch_w, ch_b, cw_w, cw_b):
    f32 = jnp.float32
    n = x1.shape[0]

    t = lambda a: jnp.transpose(a, (1, 2, 3, 0))
    x1f = t(x1).reshape(_C4 * _HW, n)
    x2f = t(x2).reshape(_C4 * _HW, n)
    x3f = t(x3).reshape(_C4 * _HW, n)
    x4f = t(x4).reshape(_C4 * _HW, n)
    dzf = t(distance).reshape(_HW, n)

    din = lambda rows: pl.BlockSpec((rows, _NB), lambda i: (0, i))
    const = lambda shape: pl.BlockSpec(shape, lambda i: (0, 0))
    out2 = pl.pallas_call(
        _cad_body,
        grid=(n // _NB,),
        in_specs=[
            din(_C4 * _HW), din(_C4 * _HW), din(_C4 * _HW), din(_C4 * _HW),
            din(_HW),
            const((_CR, _C)),
            const((_C, _CR)),
            const((_MIP, _C4)),
            const((_MIP, 1)),
            const((_MIP, 1)),
            const((_MIP, 1)),
            const((_C4, _MIP)),
            const((_C4, 1)),
            const((_C4, _MIP)),
            const((_C4, 1)),
        ],
        out_specs=pl.BlockSpec((_C * _HW, _NB), lambda i: (0, i)),
        out_shape=jax.ShapeDtypeStruct((_C * _HW, n), f32),
        compiler_params=pltpu.CompilerParams(
            dimension_semantics=("arbitrary",)),
    )(x1f, x2f, x3f, x4f, dzf, ca_w1, ca_w2, c1_w, c1_b, bn_scale, bn_shift,
      ch_w, ch_b, cw_w, cw_b)

    return jnp.transpose(out2.reshape(_C, _H, _W, n), (3, 0, 1, 2))


# speedup vs baseline: 2.6754x; 1.1157x over previous
import jax
import jax.numpy as jnp
from jax.experimental import pallas as pl
from jax.experimental.pallas import tpu as pltpu

_C4 = 16
_C = 64
_H = 16
_W = 16
_HW = _H * _W
_MIP = 8
_CR = 4
_NB = 128


def _cad_body(x1_ref, x2_ref, x3_ref, x4_ref, dz_ref, slab_ref, o_ref):
    f32 = jnp.float32

    def dot(a, b):
        return jnp.dot(a, b, preferred_element_type=f32)

    def dot_ta(a, b):
        return jax.lax.dot_general(a, b, (((0,), (0,)), ((), ())),
                                   preferred_element_type=f32)

    def dotc(w, x):
        return jax.lax.dot_general(w, x, (((1,), (0,)), ((), ())),
                                   preferred_element_type=f32)

    slab = slab_ref[...]
    w1 = slab[0:_CR, 0:_C]
    w2t = slab[_CR:2 * _CR, 0:_C]
    c1_wf = slab[8:8 + _MIP, 0:_C4]
    c1_bf = slab[8:8 + _MIP, _C4:_C4 + 1]
    ch_w = slab[0:_C4, 64:64 + _MIP]
    cw_w = slab[0:_C4, 72:72 + _MIP]
    ch_b = slab[0:_C4, 80:81]
    cw_b = slab[0:_C4, 81:82]

    nb = x1_ref.shape[-1]
    d = jax.nn.sigmoid(dz_ref[...])
    xs = (x1_ref[...], x2_ref[...], x3_ref[...], x4_ref[...])

    avg_d = jnp.mean(d, axis=0, keepdims=True)
    avgs, maxs = [], []
    for x in xs:
        x3 = x.reshape(_C4, _HW, nb)
        avgs.append(jnp.mean(x3, axis=1))
        maxs.append(jnp.max(x3 + d[None, :, :], axis=1))
    ca_avg = jnp.concatenate(avgs, axis=0) + avg_d
    ca_max = jnp.concatenate(maxs, axis=0)
    hmid = (jnp.maximum(dot(w1, ca_avg), 0.0)
            + jnp.maximum(dot(w1, ca_max), 0.0))
    ca = jax.nn.sigmoid(dot_ta(w2t, hmid))

    ip = (xs[0] + xs[1] + xs[2] + xs[3]).reshape(_C4, _HW, nb) + d[None, :, :]
    ip4 = ip.reshape(_C4, _H, _W, nb)
    ph = jnp.mean(ip4, axis=2)
    pw = jnp.mean(ip4, axis=1)
    ypool = jnp.concatenate([ph, pw], axis=1)

    y = dotc(c1_wf, ypool) + c1_bf[:, :, None]
    y = y * (jnp.clip(y + 3.0, 0.0, 6.0) * (1.0 / 6.0))

    a_h = jax.nn.sigmoid(dotc(ch_w, y[:, 0:_H, :])
                         + ch_b[:, :, None])
    a_w = jax.nn.sigmoid(dotc(cw_w, y[:, _H:, :])
                         + cw_b[:, :, None])
    g = a_h[:, :, None, :] * a_w[:, None, :, :]
    hw_a = ip4 * g
    hw_a = hw_a.reshape(_C4, _HW, nb)

    for i in range(4):
        blk = (ca[i * _C4:(i + 1) * _C4, :][:, None, :]
               * (xs[i].reshape(_C4, _HW, nb) + hw_a))
        o_ref[i * _C4 * _HW:(i + 1) * _C4 * _HW, :] = blk.reshape(_C4 * _HW, nb)


def kernel(x1, x2, x3, x4, distance, ca_w1, ca_w2, c1_w, c1_b, bn_scale,
           bn_shift, ch_w, ch_b, cw_w, cw_b):
    f32 = jnp.float32
    n = x1.shape[0]

    t = lambda a: jnp.transpose(a, (1, 2, 3, 0))
    x1f = t(x1).reshape(_C4 * _HW, n)
    x2f = t(x2).reshape(_C4 * _HW, n)
    x3f = t(x3).reshape(_C4 * _HW, n)
    x4f = t(x4).reshape(_C4 * _HW, n)
    dzf = t(distance).reshape(_HW, n)

    c1_wf = c1_w * bn_scale
    c1_bf = bn_scale * c1_b + bn_shift
    za = jnp.zeros((_MIP, _C - _C4 - 1), f32)
    left = jnp.concatenate([
        jnp.concatenate([ca_w1, ca_w2.T], axis=0),
        jnp.concatenate([c1_wf, c1_bf, za], axis=1),
    ], axis=0)
    right = jnp.concatenate(
        [ch_w, cw_w, ch_b, cw_b,
         jnp.zeros((_C4, 64 - 2 * _MIP - 2), f32)], axis=1)
    slab = jnp.concatenate([left, right], axis=1)

    din = lambda rows: pl.BlockSpec((rows, _NB), lambda i: (0, i))
    out2 = pl.pallas_call(
        _cad_body,
        grid=(n // _NB,),
        in_specs=[
            din(_C4 * _HW), din(_C4 * _HW), din(_C4 * _HW), din(_C4 * _HW),
            din(_HW),
            pl.BlockSpec((16, 128), lambda i: (0, 0)),
        ],
        out_specs=pl.BlockSpec((_C * _HW, _NB), lambda i: (0, i)),
        out_shape=jax.ShapeDtypeStruct((_C * _HW, n), f32),
        compiler_params=pltpu.CompilerParams(
            dimension_semantics=("arbitrary",)),
    )(x1f, x2f, x3f, x4f, dzf, slab)

    return jnp.transpose(out2.reshape(_C, _H, _W, n), (3, 0, 1, 2))


# speedup vs baseline: 3.1935x; 1.1937x over previous
import jax
import jax.numpy as jnp
from jax.experimental import pallas as pl
from jax.experimental.pallas import tpu as pltpu

_C4 = 16
_C = 64
_H = 16
_W = 16
_HW = _H * _W
_MIP = 8
_CR = 4
_NB = 128


def _cad_body(x1_ref, x2_ref, x3_ref, x4_ref, dz_ref, w1_ref, w2t_ref,
              c1e_ref, chwt_ref, chbt_ref, cwwt_ref, cwbt_ref, o_ref):
    f32 = jnp.float32

    def dot(a, b):
        return jnp.dot(a, b, preferred_element_type=f32)

    def dot_ta(a, b):
        return jax.lax.dot_general(a, b, (((0,), (0,)), ((), ())),
                                   preferred_element_type=f32)

    def dotc(w, x):
        return jax.lax.dot_general(w, x, (((1,), (0,)), ((), ())),
                                   preferred_element_type=f32)

    def dotc_ta(wt, x):
        return jax.lax.dot_general(wt, x, (((0,), (0,)), ((), ())),
                                   preferred_element_type=f32)

    nb = x1_ref.shape[-1]
    d = jax.nn.sigmoid(dz_ref[...])
    xs = (x1_ref[...], x2_ref[...], x3_ref[...], x4_ref[...])

    avg_d = jnp.mean(d, axis=0, keepdims=True)
    avgs, maxs = [], []
    for x in xs:
        x3 = x.reshape(_C4, _HW, nb)
        avgs.append(jnp.mean(x3, axis=1))
        maxs.append(jnp.max(x3 + d[None, :, :], axis=1))
    ca_avg = jnp.concatenate(avgs, axis=0) + avg_d
    ca_max = jnp.concatenate(maxs, axis=0)
    w1 = w1_ref[...]
    hmid = (jnp.maximum(dot(w1, ca_avg), 0.0)
            + jnp.maximum(dot(w1, ca_max), 0.0))
    ca = jax.nn.sigmoid(dot_ta(w2t_ref[...], hmid))

    ip = (xs[0] + xs[1] + xs[2] + xs[3]).reshape(_C4, _HW, nb) + d[None, :, :]
    ip4 = ip.reshape(_C4, _H, _W, nb)
    ph = jnp.mean(ip4, axis=2)
    pw = jnp.mean(ip4, axis=1)
    ypool = jnp.concatenate([ph, pw], axis=1)

    ones_row = jnp.ones((1, _H + _W, nb), f32)
    y = dotc(c1e_ref[...], jnp.concatenate([ypool, ones_row], axis=0))
    y = y * (jnp.clip(y + 3.0, 0.0, 6.0) * (1.0 / 6.0))
    y_ext = jnp.concatenate([y, ones_row], axis=0)

    chwt_ext = jnp.concatenate([chwt_ref[...], chbt_ref[...]], axis=0)
    cwwt_ext = jnp.concatenate([cwwt_ref[...], cwbt_ref[...]], axis=0)
    a_h = jax.nn.sigmoid(dotc_ta(chwt_ext, y_ext[:, 0:_H, :]))
    a_w = jax.nn.sigmoid(dotc_ta(cwwt_ext, y_ext[:, _H:, :]))
    g = a_h[:, :, None, :] * a_w[:, None, :, :]
    hw_a = ip4 * g
    hw_a = hw_a.reshape(_C4, _HW, nb)

    for i in range(4):
        blk = (ca[i * _C4:(i + 1) * _C4, :][:, None, :]
               * (xs[i].reshape(_C4, _HW, nb) + hw_a))
        o_ref[i * _C4 * _HW:(i + 1) * _C4 * _HW, :] = blk.reshape(_C4 * _HW, nb)


def kernel(x1, x2, x3, x4, distance, ca_w1, ca_w2, c1_w, c1_b, bn_scale,
           bn_shift, ch_w, ch_b, cw_w, cw_b):
    f32 = jnp.float32
    n = x1.shape[0]

    t = lambda a: jnp.transpose(a, (1, 2, 3, 0))
    x1f = t(x1).reshape(_C4 * _HW, n)
    x2f = t(x2).reshape(_C4 * _HW, n)
    x3f = t(x3).reshape(_C4 * _HW, n)
    x4f = t(x4).reshape(_C4 * _HW, n)
    dzf = t(distance).reshape(_HW, n)

    c1e = jnp.concatenate(
        [c1_w * bn_scale, bn_scale * c1_b + bn_shift], axis=1)

    din = lambda rows: pl.BlockSpec((rows, _NB), lambda i: (0, i))
    const = lambda shape: pl.BlockSpec(shape, lambda i: (0, 0))
    out2 = pl.pallas_call(
        _cad_body,
        grid=(n // _NB,),
        in_specs=[
            din(_C4 * _HW), din(_C4 * _HW), din(_C4 * _HW), din(_C4 * _HW),
            din(_HW),
            const((_CR, _C)),
            const((_CR, _C)),
            const((_MIP, _C4 + 1)),
            const((_MIP, _C4)),
            const((1, _C4)),
            const((_MIP, _C4)),
            const((1, _C4)),
        ],
        out_specs=pl.BlockSpec((_C * _HW, _NB), lambda i: (0, i)),
        out_shape=jax.ShapeDtypeStruct((_C * _HW, n), f32),
        compiler_params=pltpu.CompilerParams(
            dimension_semantics=("arbitrary",)),
    )(x1f, x2f, x3f, x4f, dzf, ca_w1, ca_w2.T, c1e, ch_w.T, ch_b.T,
      cw_w.T, cw_b.T)

    return jnp.transpose(out2.reshape(_C, _H, _W, n), (3, 0, 1, 2))


# speedup vs baseline: 3.8820x; 1.2156x over previous
import jax
import jax.numpy as jnp
from jax.experimental import pallas as pl
from jax.experimental.pallas import tpu as pltpu

_C4 = 16
_C = 64
_H = 16
_W = 16
_HW = _H * _W
_MIP = 8
_CR = 4
_NB = 128


def _cad_body(x1_ref, x2_ref, x3_ref, x4_ref, dz_ref, w1_ref, w2t_ref,
              c1w_ref, c1bt_ref, bnst_ref, bnbt_ref, chwt_ref, chbt_ref,
              cwwt_ref, cwbt_ref, o_ref):
    f32 = jnp.float32

    def dot(a, b):
        return jnp.dot(a, b, preferred_element_type=f32)

    def dot_ta(a, b):
        return jax.lax.dot_general(a, b, (((0,), (0,)), ((), ())),
                                   preferred_element_type=f32)

    def dotc(w, x):
        return jax.lax.dot_general(w, x, (((1,), (0,)), ((), ())),
                                   preferred_element_type=f32)

    def dotc_ta(wt, x):
        return jax.lax.dot_general(wt, x, (((0,), (0,)), ((), ())),
                                   preferred_element_type=f32)

    nb = x1_ref.shape[-1]
    d = jax.nn.sigmoid(dz_ref[...])
    xs = (x1_ref[...], x2_ref[...], x3_ref[...], x4_ref[...])

    avg_d = jnp.mean(d, axis=0, keepdims=True)
    avgs, maxs = [], []
    for x in xs:
        x3 = x.reshape(_C4, _HW, nb)
        avgs.append(jnp.mean(x3, axis=1))
        maxs.append(jnp.max(x3 + d[None, :, :], axis=1))
    ca_avg = jnp.concatenate(avgs, axis=0) + avg_d
    ca_max = jnp.concatenate(maxs, axis=0)
    w1 = w1_ref[...]
    hmid = (jnp.maximum(dot(w1, ca_avg), 0.0)
            + jnp.maximum(dot(w1, ca_max), 0.0))
    ca = jax.nn.sigmoid(dot_ta(w2t_ref[...], hmid))

    ip = (xs[0] + xs[1] + xs[2] + xs[3]).reshape(_C4, _HW, nb) + d[None, :, :]
    ip4 = ip.reshape(_C4, _H, _W, nb)
    ph = jnp.mean(ip4, axis=2)
    pw = jnp.mean(ip4, axis=1)
    ypool = jnp.concatenate([ph, pw], axis=1)

    ones11 = jnp.ones((1, 1), f32)

    def flip(r):
        return jax.lax.dot_general(r, ones11, (((0,), (0,)), ((), ())),
                                   preferred_element_type=f32)

    bns = bnst_ref[...]
    c1e = jnp.concatenate(
        [c1w_ref[...] * flip(bns),
         flip(bns * c1bt_ref[...] + bnbt_ref[...])], axis=1)
    ones_row = jnp.ones((1, _H + _W, nb), f32)
    y = dotc(c1e, jnp.concatenate([ypool, ones_row], axis=0))
    y = y * (jnp.clip(y + 3.0, 0.0, 6.0) * (1.0 / 6.0))
    y_ext = jnp.concatenate([y, ones_row], axis=0)

    chwt_ext = jnp.concatenate([chwt_ref[...], chbt_ref[...]], axis=0)
    cwwt_ext = jnp.concatenate([cwwt_ref[...], cwbt_ref[...]], axis=0)
    a_h = jax.nn.sigmoid(dotc_ta(chwt_ext, y_ext[:, 0:_H, :]))
    a_w = jax.nn.sigmoid(dotc_ta(cwwt_ext, y_ext[:, _H:, :]))
    g = a_h[:, :, None, :] * a_w[:, None, :, :]
    hw_a = ip4 * g
    hw_a = hw_a.reshape(_C4, _HW, nb)

    for i in range(4):
        blk = (ca[i * _C4:(i + 1) * _C4, :][:, None, :]
               * (xs[i].reshape(_C4, _HW, nb) + hw_a))
        o_ref[i * _C4 * _HW:(i + 1) * _C4 * _HW, :] = blk.reshape(_C4 * _HW, nb)


def kernel(x1, x2, x3, x4, distance, ca_w1, ca_w2, c1_w, c1_b, bn_scale,
           bn_shift, ch_w, ch_b, cw_w, cw_b):
    f32 = jnp.float32
    n = x1.shape[0]

    t = lambda a: jnp.transpose(a, (1, 2, 3, 0))
    x1f = t(x1).reshape(_C4 * _HW, n)
    x2f = t(x2).reshape(_C4 * _HW, n)
    x3f = t(x3).reshape(_C4 * _HW, n)
    x4f = t(x4).reshape(_C4 * _HW, n)
    dzf = t(distance).reshape(_HW, n)

    din = lambda rows: pl.BlockSpec((rows, _NB), lambda i: (0, i))
    const = lambda shape: pl.BlockSpec(shape, lambda i: (0, 0))
    out2 = pl.pallas_call(
        _cad_body,
        grid=(n // _NB,),
        in_specs=[
            din(_C4 * _HW), din(_C4 * _HW), din(_C4 * _HW), din(_C4 * _HW),
            din(_HW),
            const((_CR, _C)),
            const((_CR, _C)),
            const((_MIP, _C4)),
            const((1, _MIP)),
            const((1, _MIP)),
            const((1, _MIP)),
            const((_MIP, _C4)),
            const((1, _C4)),
            const((_MIP, _C4)),
            const((1, _C4)),
        ],
        out_specs=pl.BlockSpec((_C * _HW, _NB), lambda i: (0, i)),
        out_shape=jax.ShapeDtypeStruct((_C * _HW, n), f32),
        compiler_params=pltpu.CompilerParams(
            dimension_semantics=("arbitrary",)),
    )(x1f, x2f, x3f, x4f, dzf, ca_w1, ca_w2.T, c1_w, c1_b.T, bn_scale.T,
      bn_shift.T, ch_w.T, ch_b.T, cw_w.T, cw_b.T)

    return jnp.transpose(out2.reshape(_C, _H, _W, n), (3, 0, 1, 2))
